# Initial kernel scaffold; baseline (speedup 1.0000x reference)
import jax
import jax.numpy as jnp
from jax.experimental import pallas as pl


def kernel(x, edge_index, W1, att_src1, att_dst1, b1, W2, att_src2, att_dst2, b2):
    raise NotImplementedError("write your pallas kernel here")


# speedup vs baseline: 18.3342x; 18.3342x over previous
import dataclasses
import functools

import jax
import jax.numpy as jnp
from jax import lax
from jax.experimental import pallas as pl
from jax.experimental.pallas import tpu as pltpu
from jax.experimental.pallas import tpu_sc as plsc

N = 10000
E = 320000
F_IN = 128
HID = 128
NCLS = 40
NCLSP = 48

NC = 2
NS = 16
L = 16
NW = NC * NS
EPW = E // NW
NPAD = 10240
RPT = NPAD // NS
CH = 16
IB = 2000
NIB = EPW // IB
ICH = IB // CH
ZR = 16


def _matmul(x, w):
    m, k = x.shape
    _, n = w.shape
    bm = 2000

    def body(x_ref, w_ref, o_ref):
        o_ref[...] = jnp.dot(x_ref[...], w_ref[...],
                             preferred_element_type=jnp.float32)

    return pl.pallas_call(
        body,
        grid=(m // bm,),
        in_specs=[pl.BlockSpec((bm, k), lambda i: (i, 0)),
                  pl.BlockSpec((k, n), lambda i: (0, 0))],
        out_specs=pl.BlockSpec((bm, n), lambda i: (i, 0)),
        out_shape=jax.ShapeDtypeStruct((m, n), jnp.float32),
    )(x, w)


def _make_edge_pass(D):
    W = D + L
    mesh = plsc.VectorSubcoreMesh(core_axis_name="c", subcore_axis_name="s")
    cp = pltpu.CompilerParams()
    if "needs_layout_passes" in pltpu.CompilerParams.__dataclass_fields__:
        cp = dataclasses.replace(cp, needs_layout_passes=False)
    if "use_tc_tiling_on_sc" in pltpu.CompilerParams.__dataclass_fields__:
        cp = dataclasses.replace(cp, use_tc_tiling_on_sc=False)

    @functools.partial(
        pl.kernel,
        compiler_params=cp,
        out_type=jax.ShapeDtypeStruct((NC, NPAD, W), jnp.float32),
        mesh=mesh,
        scratch_types=[
            pltpu.VMEM((IB,), jnp.int32),
            pltpu.VMEM((IB,), jnp.int32),
            pltpu.VMEM((N,), jnp.float32),
            pltpu.VMEM((N,), jnp.float32),
            pltpu.VMEM((L,), jnp.float32),
            pltpu.VMEM((L,), jnp.float32),
            pltpu.VMEM((CH, D), jnp.float32),
            pltpu.VMEM((CH, W), jnp.float32),
            pltpu.VMEM((ZR, W), jnp.float32),
            pltpu.VMEM_SHARED((NPAD, W), jnp.float32),
            pltpu.SemaphoreType.DMA,
        ],
    )
    def edge_pass(asrc_hbm, adst_hbm, c_hbm, src_hbm, dst_hbm, h_hbm, out_hbm,
                  src_v, dst_v, asrc_v, adst_v, c_v, p_v, rows_v, stage_v,
                  zero_v, acc_sh, sem):
        cid = lax.axis_index("c")
        sid = lax.axis_index("s")
        wid = sid * NC + cid
        ebase = wid * EPW

        pltpu.sync_copy(asrc_hbm, asrc_v)
        pltpu.sync_copy(adst_hbm, adst_v)
        pltpu.sync_copy(c_hbm, c_v)

        zvec = jnp.zeros((L,), jnp.float32)

        @pl.loop(0, ZR)
        def _(r):
            for q in range(W // L):
                zero_v[r, pl.ds(q * L, L)] = zvec

        @pl.loop(0, RPT // ZR)
        def _(k):
            pltpu.sync_copy(zero_v, acc_sh.at[pl.ds(sid * RPT + k * ZR, ZR)])

        plsc.subcore_barrier()

        cvec = c_v[...]

        @pl.loop(0, NIB)
        def _(bi):
            pltpu.sync_copy(src_hbm.at[pl.ds(ebase + bi * IB, IB)], src_v)
            pltpu.sync_copy(dst_hbm.at[pl.ds(ebase + bi * IB, IB)], dst_v)

            @pl.loop(0, ICH)
            def _(ci):
                srcs = src_v[pl.ds(ci * CH, CH)]
                dsts = dst_v[pl.ds(ci * CH, CH)]
                a_s = plsc.load_gather(asrc_v, [srcs])
                a_d = plsc.load_gather(adst_v, [dsts])
                s = a_s + a_d
                alpha = jnp.where(s > 0, s, 0.2 * s)
                p = jnp.exp(alpha - cvec)
                pltpu.async_copy(h_hbm.at[srcs], rows_v, sem).wait()
                for j in range(CH):
                    pj = p[jnp.full((L,), j, jnp.int32)]
                    stage_v[j, pl.ds(0, L)] = pj
                    for q in range(D // L):
                        stage_v[j, pl.ds(L + q * L, L)] = \
                            rows_v[j, pl.ds(q * L, L)] * pj
                pltpu.sync_copy(stage_v, acc_sh.at[dsts], add=True)

        plsc.subcore_barrier()
        pltpu.sync_copy(acc_sh.at[pl.ds(sid * RPT, RPT)],
                        out_hbm.at[cid, pl.ds(sid * RPT, RPT)])

    return edge_pass


_edge_pass_hid = _make_edge_pass(HID)
_edge_pass_cls = _make_edge_pass(NCLSP)


def _lrelu(s):
    return jnp.where(s > 0, s, 0.2 * s)


def kernel(x, edge_index, W1, att_src1, att_dst1, b1,
           W2, att_src2, att_dst2, b2):
    src = edge_index[0]
    dst = edge_index[1]

    h1 = _matmul(x, W1)
    a_s1 = h1 @ att_src1
    a_d1 = h1 @ att_dst1
    c1 = jnp.max(a_s1) + jnp.max(a_d1)
    p_self1 = jnp.exp(_lrelu(a_s1 + a_d1) - c1)
    acc1 = _edge_pass_hid(a_s1, a_d1, jnp.full((L,), c1, jnp.float32),
                          src, dst, h1)[:, :N]
    denom1 = acc1[0, :, 0] + acc1[1, :, 0] + p_self1
    num1 = acc1[0, :, L:] + acc1[1, :, L:] + p_self1[:, None] * h1
    r1 = jax.nn.relu(num1 / denom1[:, None] + b1)

    W2p = jnp.pad(W2, ((0, 0), (0, NCLSP - NCLS)))
    h2 = _matmul(r1, W2p)
    a_s2 = h2[:, :NCLS] @ att_src2
    a_d2 = h2[:, :NCLS] @ att_dst2
    c2 = jnp.max(a_s2) + jnp.max(a_d2)
    p_self2 = jnp.exp(_lrelu(a_s2 + a_d2) - c2)
    acc2 = _edge_pass_cls(a_s2, a_d2, jnp.full((L,), c2, jnp.float32),
                          src, dst, h2)[:, :N]
    denom2 = acc2[0, :, 0] + acc2[1, :, 0] + p_self2
    num2 = (acc2[0, :, L:L + NCLS] + acc2[1, :, L:L + NCLS]
            + p_self2[:, None] * h2[:, :NCLS])
    return num2 / denom2[:, None] + b2


# speedup vs baseline: 34.2530x; 1.8683x over previous
import dataclasses
import functools

import jax
import jax.numpy as jnp
from jax import lax
from jax.experimental import pallas as pl
from jax.experimental.pallas import tpu as pltpu
from jax.experimental.pallas import tpu_sc as plsc

N = 10000
E = 320000
F_IN = 128
HID = 128
NCLS = 40
NCLSP = 48

NC = 2
NS = 16
L = 16
NW = NC * NS
EPW = E // NW
NPAD = 10240
RPT = NPAD // NS
CH = 16
IB = 2000
NIB = EPW // IB
ICH = IB // CH
ZR = 16


def _matmul(x, w):
    m, k = x.shape
    _, n = w.shape
    bm = 2000

    def body(x_ref, w_ref, o_ref):
        o_ref[...] = jnp.dot(x_ref[...], w_ref[...],
                             preferred_element_type=jnp.float32)

    return pl.pallas_call(
        body,
        grid=(m // bm,),
        in_specs=[pl.BlockSpec((bm, k), lambda i: (i, 0)),
                  pl.BlockSpec((k, n), lambda i: (0, 0))],
        out_specs=pl.BlockSpec((bm, n), lambda i: (i, 0)),
        out_shape=jax.ShapeDtypeStruct((m, n), jnp.float32),
    )(x, w)


def _make_edge_pass(D):
    W = D + L
    mesh = plsc.VectorSubcoreMesh(core_axis_name="c", subcore_axis_name="s")
    cp = pltpu.CompilerParams()
    if "needs_layout_passes" in pltpu.CompilerParams.__dataclass_fields__:
        cp = dataclasses.replace(cp, needs_layout_passes=False)
    if "use_tc_tiling_on_sc" in pltpu.CompilerParams.__dataclass_fields__:
        cp = dataclasses.replace(cp, use_tc_tiling_on_sc=False)

    @functools.partial(
        pl.kernel,
        compiler_params=cp,
        out_type=jax.ShapeDtypeStruct((NC, NPAD, W), jnp.float32),
        mesh=mesh,
        scratch_types=[
            pltpu.VMEM((IB,), jnp.int32),
            pltpu.VMEM((IB,), jnp.int32),
            pltpu.VMEM((N,), jnp.float32),
            pltpu.VMEM((N,), jnp.float32),
            pltpu.VMEM((L,), jnp.float32),
            pltpu.VMEM((CH, D), jnp.float32),
            pltpu.VMEM((CH, D), jnp.float32),
            pltpu.VMEM((CH, W), jnp.float32),
            pltpu.VMEM((CH, W), jnp.float32),
            pltpu.VMEM((ZR, W), jnp.float32),
            pltpu.VMEM_SHARED((NPAD, W), jnp.float32),
            pltpu.SemaphoreType.DMA,
            pltpu.SemaphoreType.DMA,
            pltpu.SemaphoreType.DMA,
            pltpu.SemaphoreType.DMA,
        ],
    )
    def edge_pass(asrc_hbm, adst_hbm, c_hbm, src_hbm, dst_hbm, h_hbm, out_hbm,
                  src_v, dst_v, asrc_v, adst_v, c_v, rows0_v, rows1_v,
                  stage0_v, stage1_v, zero_v, acc_sh, gsem0, gsem1,
                  ssem0, ssem1):
        cid = lax.axis_index("c")
        sid = lax.axis_index("s")
        wid = sid * NC + cid
        ebase = wid * EPW

        pltpu.sync_copy(asrc_hbm, asrc_v)
        pltpu.sync_copy(adst_hbm, adst_v)
        pltpu.sync_copy(c_hbm, c_v)

        zvec = jnp.zeros((L,), jnp.float32)

        @pl.loop(0, ZR)
        def _(r):
            for q in range(W // L):
                zero_v[r, pl.ds(q * L, L)] = zvec

        @pl.loop(0, RPT // ZR)
        def _(k):
            pltpu.sync_copy(zero_v, acc_sh.at[pl.ds(sid * RPT + k * ZR, ZR)])

        plsc.subcore_barrier()

        cvec = c_v[...]

        def step(ci, rows_v, stage_v, gsem, ssem):
            srcs = src_v[pl.ds(ci * CH, CH)]
            dsts = dst_v[pl.ds(ci * CH, CH)]
            pltpu.make_async_copy(h_hbm.at[srcs], rows_v, gsem).wait()
            a_s = plsc.load_gather(asrc_v, [srcs])
            a_d = plsc.load_gather(adst_v, [dsts])
            s = a_s + a_d
            alpha = jnp.where(s > 0, s, 0.2 * s)
            p = jnp.exp(alpha - cvec)

            @pl.when(ci >= 2)
            def _():
                pltpu.make_async_copy(
                    stage_v, acc_sh.at[dsts], ssem).wait()

            for j in range(CH):
                pj = p[jnp.full((L,), j, jnp.int32)]
                stage_v[j, pl.ds(0, L)] = pj
                for q in range(D // L):
                    stage_v[j, pl.ds(L + q * L, L)] = \
                        rows_v[j, pl.ds(q * L, L)] * pj
            scat = pltpu.make_async_copy(stage_v, acc_sh.at[dsts], ssem)
            scat.start(add=True)

            @pl.when(ci + 2 < ICH)
            def _():
                srcs2 = src_v[pl.ds((ci + 2) * CH, CH)]
                pltpu.make_async_copy(h_hbm.at[srcs2], rows_v, gsem).start()

        @pl.loop(0, NIB)
        def _(bi):
            pltpu.sync_copy(src_hbm.at[pl.ds(ebase + bi * IB, IB)], src_v)
            pltpu.sync_copy(dst_hbm.at[pl.ds(ebase + bi * IB, IB)], dst_v)

            pltpu.make_async_copy(
                h_hbm.at[src_v[pl.ds(0, CH)]], rows0_v, gsem0).start()
            pltpu.make_async_copy(
                h_hbm.at[src_v[pl.ds(CH, CH)]], rows1_v, gsem1).start()

            @pl.loop(0, ICH)
            def _(ci):
                @pl.when(ci % 2 == 0)
                def _():
                    step(ci, rows0_v, stage0_v, gsem0, ssem0)

                @pl.when(ci % 2 == 1)
                def _():
                    step(ci, rows1_v, stage1_v, gsem1, ssem1)

            dd = dst_v[pl.ds(0, CH)]
            pltpu.make_async_copy(stage0_v, acc_sh.at[dd], ssem0).wait()
            pltpu.make_async_copy(stage1_v, acc_sh.at[dd], ssem1).wait()

        plsc.subcore_barrier()
        pltpu.sync_copy(acc_sh.at[pl.ds(sid * RPT, RPT)],
                        out_hbm.at[cid, pl.ds(sid * RPT, RPT)])

    return edge_pass


_edge_pass_hid = _make_edge_pass(HID)
_edge_pass_cls = _make_edge_pass(NCLSP)


def _lrelu(s):
    return jnp.where(s > 0, s, 0.2 * s)


def kernel(x, edge_index, W1, att_src1, att_dst1, b1,
           W2, att_src2, att_dst2, b2):
    src = edge_index[0]
    dst = edge_index[1]

    h1 = _matmul(x, W1)
    a_s1 = h1 @ att_src1
    a_d1 = h1 @ att_dst1
    c1 = jnp.max(a_s1) + jnp.max(a_d1)
    p_self1 = jnp.exp(_lrelu(a_s1 + a_d1) - c1)
    acc1 = _edge_pass_hid(a_s1, a_d1, jnp.full((L,), c1, jnp.float32),
                          src, dst, h1)[:, :N]
    denom1 = acc1[0, :, 0] + acc1[1, :, 0] + p_self1
    num1 = acc1[0, :, L:] + acc1[1, :, L:] + p_self1[:, None] * h1
    r1 = jax.nn.relu(num1 / denom1[:, None] + b1)

    W2p = jnp.pad(W2, ((0, 0), (0, NCLSP - NCLS)))
    h2 = _matmul(r1, W2p)
    a_s2 = h2[:, :NCLS] @ att_src2
    a_d2 = h2[:, :NCLS] @ att_dst2
    c2 = jnp.max(a_s2) + jnp.max(a_d2)
    p_self2 = jnp.exp(_lrelu(a_s2 + a_d2) - c2)
    acc2 = _edge_pass_cls(a_s2, a_d2, jnp.full((L,), c2, jnp.float32),
                          src, dst, h2)[:, :N]
    denom2 = acc2[0, :, 0] + acc2[1, :, 0] + p_self2
    num2 = (acc2[0, :, L:L + NCLS] + acc2[1, :, L:L + NCLS]
            + p_self2[:, None] * h2[:, :NCLS])
    return num2 / denom2[:, None] + b2


# speedup vs baseline: 42.5674x; 1.2427x over previous
import dataclasses
import functools

import jax
import jax.numpy as jnp
from jax import lax
from jax.experimental import pallas as pl
from jax.experimental.pallas import tpu as pltpu
from jax.experimental.pallas import tpu_sc as plsc

N = 10000
E = 320000
F_IN = 128
HID = 128
NCLS = 40
NCLSP = 48

NC = 2
NS = 16
L = 16
NW = NC * NS
EPW = E // NW
NPAD = 10240
RPT = NPAD // NS
IB = 2000
NIB = EPW // IB
ZR = 16


def _matmul(x, w):
    m, k = x.shape
    _, n = w.shape
    bm = 2000

    def body(x_ref, w_ref, o_ref):
        o_ref[...] = jnp.dot(x_ref[...], w_ref[...],
                             preferred_element_type=jnp.float32)

    return pl.pallas_call(
        body,
        grid=(m // bm,),
        in_specs=[pl.BlockSpec((bm, k), lambda i: (i, 0)),
                  pl.BlockSpec((k, n), lambda i: (0, 0))],
        out_specs=pl.BlockSpec((bm, n), lambda i: (i, 0)),
        out_shape=jax.ShapeDtypeStruct((m, n), jnp.float32),
    )(x, w)


def _make_edge_pass(D, CH):
    W = D + L
    ICH = IB // CH
    mesh = plsc.VectorSubcoreMesh(core_axis_name="c", subcore_axis_name="s")
    cp = pltpu.CompilerParams()
    if "needs_layout_passes" in pltpu.CompilerParams.__dataclass_fields__:
        cp = dataclasses.replace(cp, needs_layout_passes=False)
    if "use_tc_tiling_on_sc" in pltpu.CompilerParams.__dataclass_fields__:
        cp = dataclasses.replace(cp, use_tc_tiling_on_sc=False)

    @functools.partial(
        pl.kernel,
        compiler_params=cp,
        out_type=jax.ShapeDtypeStruct((NC, NPAD, W), jnp.float32),
        mesh=mesh,
        scratch_types=[
            pltpu.VMEM((IB,), jnp.int32),
            pltpu.VMEM((IB,), jnp.int32),
            pltpu.VMEM((N,), jnp.float32),
            pltpu.VMEM((N,), jnp.float32),
            pltpu.VMEM((L,), jnp.float32),
            pltpu.VMEM((CH, D), jnp.float32),
            pltpu.VMEM((CH, D), jnp.float32),
            pltpu.VMEM((CH, W), jnp.float32),
            pltpu.VMEM((CH, W), jnp.float32),
            pltpu.VMEM((CH,), jnp.int32),
            pltpu.VMEM((CH,), jnp.int32),
            pltpu.VMEM((ZR, W), jnp.float32),
            pltpu.VMEM_SHARED((NPAD, W), jnp.float32),
            pltpu.SemaphoreType.DMA,
            pltpu.SemaphoreType.DMA,
            pltpu.SemaphoreType.DMA,
            pltpu.SemaphoreType.DMA,
        ],
    )
    def edge_pass(asrc_hbm, adst_hbm, c_hbm, src_hbm, dst_hbm, h_hbm, out_hbm,
                  src_v, dst_v, asrc_v, adst_v, c_v, rows0_v, rows1_v,
                  stage0_v, stage1_v, dstb0_v, dstb1_v, zero_v, acc_sh,
                  gsem0, gsem1, ssem0, ssem1):
        cid = lax.axis_index("c")
        sid = lax.axis_index("s")
        wid = sid * NC + cid
        ebase = wid * EPW

        pltpu.sync_copy(asrc_hbm, asrc_v)
        pltpu.sync_copy(adst_hbm, adst_v)
        pltpu.sync_copy(c_hbm, c_v)

        zvec = jnp.zeros((L,), jnp.float32)

        @pl.loop(0, ZR)
        def _(r):
            for q in range(W // L):
                zero_v[r, pl.ds(q * L, L)] = zvec

        @pl.loop(0, RPT // ZR)
        def _(k):
            pltpu.sync_copy(zero_v, acc_sh.at[pl.ds(sid * RPT + k * ZR, ZR)])

        plsc.subcore_barrier()

        cvec = c_v[...]

        def step(ci, rows_v, stage_v, dstb_v, gsem, ssem):
            base = ci * CH
            idx = src_v.at[pl.ds(base, CH)]
            pltpu.make_async_copy(h_hbm.at[idx], rows_v, gsem).wait()

            @pl.when(ci >= 2)
            def _():
                pltpu.make_async_copy(
                    stage_v, acc_sh.at[dstb_v], ssem).wait()

            for sub in range(CH // L):
                srcs = src_v[pl.ds(base + sub * L, L)]
                dsts = dst_v[pl.ds(base + sub * L, L)]
                dstb_v[pl.ds(sub * L, L)] = dsts
                a_s = plsc.load_gather(asrc_v, [srcs])
                a_d = plsc.load_gather(adst_v, [dsts])
                s = a_s + a_d
                alpha = jnp.where(s > 0, s, 0.2 * s)
                p = jnp.exp(alpha - cvec)
                for j in range(L):
                    r = sub * L + j
                    pj = p[jnp.full((L,), j, jnp.int32)]
                    stage_v[r, pl.ds(0, L)] = pj
                    for q in range(D // L):
                        stage_v[r, pl.ds(L + q * L, L)] = \
                            rows_v[r, pl.ds(q * L, L)] * pj
            scat = pltpu.make_async_copy(stage_v, acc_sh.at[dstb_v], ssem)
            scat.start(add=True)

            @pl.when(ci + 2 < ICH)
            def _():
                idx2 = src_v.at[pl.ds((ci + 2) * CH, CH)]
                pltpu.make_async_copy(h_hbm.at[idx2], rows_v, gsem).start()

        @pl.loop(0, NIB)
        def _(bi):
            pltpu.sync_copy(src_hbm.at[pl.ds(ebase + bi * IB, IB)], src_v)
            pltpu.sync_copy(dst_hbm.at[pl.ds(ebase + bi * IB, IB)], dst_v)

            pltpu.make_async_copy(
                h_hbm.at[src_v.at[pl.ds(0, CH)]], rows0_v, gsem0).start()
            pltpu.make_async_copy(
                h_hbm.at[src_v.at[pl.ds(CH, CH)]], rows1_v, gsem1).start()

            @pl.loop(0, ICH)
            def _(ci):
                @pl.when(ci % 2 == 0)
                def _():
                    step(ci, rows0_v, stage0_v, dstb0_v, gsem0, ssem0)

                @pl.when(ci % 2 == 1)
                def _():
                    step(ci, rows1_v, stage1_v, dstb1_v, gsem1, ssem1)

            pltpu.make_async_copy(stage0_v, acc_sh.at[dstb0_v], ssem0).wait()
            pltpu.make_async_copy(stage1_v, acc_sh.at[dstb1_v], ssem1).wait()

        plsc.subcore_barrier()
        pltpu.sync_copy(acc_sh.at[pl.ds(sid * RPT, RPT)],
                        out_hbm.at[cid, pl.ds(sid * RPT, RPT)])

    return edge_pass


_edge_pass_hid = _make_edge_pass(HID, 16)
_edge_pass_cls = _make_edge_pass(NCLSP, 80)


def _lrelu(s):
    return jnp.where(s > 0, s, 0.2 * s)


def kernel(x, edge_index, W1, att_src1, att_dst1, b1,
           W2, att_src2, att_dst2, b2):
    src = edge_index[0]
    dst = edge_index[1]

    h1 = _matmul(x, W1)
    a_s1 = h1 @ att_src1
    a_d1 = h1 @ att_dst1
    c1 = jnp.max(a_s1) + jnp.max(a_d1)
    p_self1 = jnp.exp(_lrelu(a_s1 + a_d1) - c1)
    acc1 = _edge_pass_hid(a_s1, a_d1, jnp.full((L,), c1, jnp.float32),
                          src, dst, h1)[:, :N]
    denom1 = acc1[0, :, 0] + acc1[1, :, 0] + p_self1
    num1 = acc1[0, :, L:] + acc1[1, :, L:] + p_self1[:, None] * h1
    r1 = jax.nn.relu(num1 / denom1[:, None] + b1)

    W2p = jnp.pad(W2, ((0, 0), (0, NCLSP - NCLS)))
    h2 = _matmul(r1, W2p)
    a_s2 = h2[:, :NCLS] @ att_src2
    a_d2 = h2[:, :NCLS] @ att_dst2
    c2 = jnp.max(a_s2) + jnp.max(a_d2)
    p_self2 = jnp.exp(_lrelu(a_s2 + a_d2) - c2)
    acc2 = _edge_pass_cls(a_s2, a_d2, jnp.full((L,), c2, jnp.float32),
                          src, dst, h2)[:, :N]
    denom2 = acc2[0, :, 0] + acc2[1, :, 0] + p_self2
    num2 = (acc2[0, :, L:L + NCLS] + acc2[1, :, L:L + NCLS]
            + p_self2[:, None] * h2[:, :NCLS])
    return num2 / denom2[:, None] + b2


# speedup vs baseline: 54.7784x; 1.2869x over previous
import dataclasses
import functools

import jax
import jax.numpy as jnp
from jax import lax
from jax.experimental import pallas as pl
from jax.experimental.pallas import tpu as pltpu
from jax.experimental.pallas import tpu_sc as plsc

N = 10000
E = 320000
F_IN = 128
HID = 128
NCLS = 40
NCLSP = 48

NC = 2
NS = 16
L = 16
NW = NC * NS
EPW = E // NW
NPAD = 10240
RPT = NPAD // NS
IB = 2000
NIB = EPW // IB
ZR = 16


def _matmul(x, w):
    m, k = x.shape
    _, n = w.shape
    bm = 2000

    def body(x_ref, w_ref, o_ref):
        o_ref[...] = jnp.dot(x_ref[...], w_ref[...],
                             preferred_element_type=jnp.float32)

    return pl.pallas_call(
        body,
        grid=(m // bm,),
        in_specs=[pl.BlockSpec((bm, k), lambda i: (i, 0)),
                  pl.BlockSpec((k, n), lambda i: (0, 0))],
        out_specs=pl.BlockSpec((bm, n), lambda i: (i, 0)),
        out_shape=jax.ShapeDtypeStruct((m, n), jnp.float32),
    )(x, w)


def _make_edge_pass(D, CH):
    W = D + L
    ICH = IB // CH
    mesh = plsc.VectorSubcoreMesh(core_axis_name="c", subcore_axis_name="s")
    cp = pltpu.CompilerParams()
    if "needs_layout_passes" in pltpu.CompilerParams.__dataclass_fields__:
        cp = dataclasses.replace(cp, needs_layout_passes=False)
    if "use_tc_tiling_on_sc" in pltpu.CompilerParams.__dataclass_fields__:
        cp = dataclasses.replace(cp, use_tc_tiling_on_sc=False)

    @functools.partial(
        pl.kernel,
        compiler_params=cp,
        out_type=jax.ShapeDtypeStruct((NC, NPAD, W), jnp.float32),
        mesh=mesh,
        scratch_types=[
            pltpu.VMEM((IB,), jnp.int32),
            pltpu.VMEM((IB,), jnp.int32),
            pltpu.VMEM((N,), jnp.float32),
            pltpu.VMEM((N,), jnp.float32),
            pltpu.VMEM((L,), jnp.float32),
            pltpu.VMEM((CH, D), jnp.float32),
            pltpu.VMEM((CH, D), jnp.float32),
            pltpu.VMEM((CH, W), jnp.float32),
            pltpu.VMEM((CH, W), jnp.float32),
            pltpu.VMEM((CH,), jnp.int32),
            pltpu.VMEM((CH,), jnp.int32),
            pltpu.VMEM((ZR, W), jnp.float32),
            pltpu.VMEM_SHARED((NPAD, W), jnp.float32),
            pltpu.SemaphoreType.DMA,
            pltpu.SemaphoreType.DMA,
            pltpu.SemaphoreType.DMA,
            pltpu.SemaphoreType.DMA,
        ],
    )
    def edge_pass(asrc_hbm, adst_hbm, c_hbm, src_hbm, dst_hbm, h_hbm, out_hbm,
                  src_v, dst_v, asrc_v, adst_v, c_v, rows0_v, rows1_v,
                  stage0_v, stage1_v, dstb0_v, dstb1_v, zero_v, acc_sh,
                  gsem0, gsem1, ssem0, ssem1):
        cid = lax.axis_index("c")
        sid = lax.axis_index("s")
        wid = sid * NC + cid
        ebase = wid * EPW

        pltpu.sync_copy(asrc_hbm, asrc_v)
        pltpu.sync_copy(adst_hbm, adst_v)
        pltpu.sync_copy(c_hbm, c_v)

        zvec = jnp.zeros((L,), jnp.float32)

        @pl.loop(0, ZR)
        def _(r):
            for q in range(W // L):
                zero_v[r, pl.ds(q * L, L)] = zvec

        @pl.loop(0, RPT // ZR)
        def _(k):
            pltpu.sync_copy(zero_v, acc_sh.at[pl.ds(sid * RPT + k * ZR, ZR)])

        plsc.subcore_barrier()

        cvec = c_v[...]

        def step(ci, rows_v, stage_v, dstb_v, gsem, ssem):
            base = ci * CH
            idx = src_v.at[pl.ds(base, CH)]
            pltpu.make_async_copy(h_hbm.at[idx], rows_v, gsem).wait()

            @pl.when(ci >= 2)
            def _():
                pltpu.make_async_copy(
                    stage_v, acc_sh.at[dstb_v], ssem).wait()

            for sub in range(CH // L):
                srcs = src_v[pl.ds(base + sub * L, L)]
                dsts = dst_v[pl.ds(base + sub * L, L)]
                dstb_v[pl.ds(sub * L, L)] = dsts
                a_s = plsc.load_gather(asrc_v, [srcs])
                a_d = plsc.load_gather(adst_v, [dsts])
                s = a_s + a_d
                alpha = jnp.where(s > 0, s, 0.2 * s)
                p = jnp.exp(alpha - cvec)
                for j in range(L):
                    r = sub * L + j
                    pj = p[jnp.full((L,), j, jnp.int32)]
                    stage_v[r, pl.ds(0, L)] = pj
                    for q in range(D // L):
                        stage_v[r, pl.ds(L + q * L, L)] = \
                            rows_v[r, pl.ds(q * L, L)] * pj
            scat = pltpu.make_async_copy(stage_v, acc_sh.at[dstb_v], ssem)
            scat.start(add=True)

            @pl.when(ci + 2 < ICH)
            def _():
                idx2 = src_v.at[pl.ds((ci + 2) * CH, CH)]
                pltpu.make_async_copy(h_hbm.at[idx2], rows_v, gsem).start()

        @pl.loop(0, NIB)
        def _(bi):
            pltpu.sync_copy(src_hbm.at[pl.ds(ebase + bi * IB, IB)], src_v)
            pltpu.sync_copy(dst_hbm.at[pl.ds(ebase + bi * IB, IB)], dst_v)

            pltpu.make_async_copy(
                h_hbm.at[src_v.at[pl.ds(0, CH)]], rows0_v, gsem0).start()
            pltpu.make_async_copy(
                h_hbm.at[src_v.at[pl.ds(CH, CH)]], rows1_v, gsem1).start()

            @pl.loop(0, ICH)
            def _(ci):
                @pl.when(ci % 2 == 0)
                def _():
                    step(ci, rows0_v, stage0_v, dstb0_v, gsem0, ssem0)

                @pl.when(ci % 2 == 1)
                def _():
                    step(ci, rows1_v, stage1_v, dstb1_v, gsem1, ssem1)

            pltpu.make_async_copy(stage0_v, acc_sh.at[dstb0_v], ssem0).wait()
            pltpu.make_async_copy(stage1_v, acc_sh.at[dstb1_v], ssem1).wait()

        plsc.subcore_barrier()
        pltpu.sync_copy(acc_sh.at[pl.ds(sid * RPT, RPT)],
                        out_hbm.at[cid, pl.ds(sid * RPT, RPT)])

    return edge_pass


_edge_pass_cls = _make_edge_pass(NCLSP, 80)

_mesh = plsc.VectorSubcoreMesh(core_axis_name="c", subcore_axis_name="s")
_cp = pltpu.CompilerParams()
if "needs_layout_passes" in pltpu.CompilerParams.__dataclass_fields__:
    _cp = dataclasses.replace(_cp, needs_layout_passes=False)
if "use_tc_tiling_on_sc" in pltpu.CompilerParams.__dataclass_fields__:
    _cp = dataclasses.replace(_cp, use_tc_tiling_on_sc=False)

ACH = 80


@functools.partial(
    pl.kernel,
    compiler_params=_cp,
    out_type=[jax.ShapeDtypeStruct((E,), jnp.float32),
              jax.ShapeDtypeStruct((NC, NPAD, L), jnp.float32)],
    mesh=_mesh,
    scratch_types=[
        pltpu.VMEM((IB,), jnp.int32),
        pltpu.VMEM((IB,), jnp.int32),
        pltpu.VMEM((N,), jnp.float32),
        pltpu.VMEM((N,), jnp.float32),
        pltpu.VMEM((L,), jnp.float32),
        pltpu.VMEM((IB,), jnp.float32),
        pltpu.VMEM((ACH, L), jnp.float32),
        pltpu.VMEM((ACH,), jnp.int32),
        pltpu.VMEM((ZR, L), jnp.float32),
        pltpu.VMEM_SHARED((NPAD, L), jnp.float32),
    ],
)
def _alpha_pass(asrc_hbm, adst_hbm, c_hbm, src_hbm, dst_hbm, p_hbm, den_hbm,
                src_v, dst_v, asrc_v, adst_v, c_v, pbuf_v, sden_v, dstb_v,
                zero_v, acc_sh):
    cid = lax.axis_index("c")
    sid = lax.axis_index("s")
    wid = sid * NC + cid
    ebase = wid * EPW

    pltpu.sync_copy(asrc_hbm, asrc_v)
    pltpu.sync_copy(adst_hbm, adst_v)
    pltpu.sync_copy(c_hbm, c_v)

    zvec = jnp.zeros((L,), jnp.float32)

    @pl.loop(0, ZR)
    def _(r):
        zero_v[r, pl.ds(0, L)] = zvec

    @pl.loop(0, RPT // ZR)
    def _(k):
        pltpu.sync_copy(zero_v, acc_sh.at[pl.ds(sid * RPT + k * ZR, ZR)])

    plsc.subcore_barrier()

    cvec = c_v[...]

    @pl.loop(0, NIB)
    def _(bi):
        pltpu.sync_copy(src_hbm.at[pl.ds(ebase + bi * IB, IB)], src_v)
        pltpu.sync_copy(dst_hbm.at[pl.ds(ebase + bi * IB, IB)], dst_v)

        @pl.loop(0, IB // ACH)
        def _(ci):
            base = ci * ACH
            for sub in range(ACH // L):
                srcs = src_v[pl.ds(base + sub * L, L)]
                dsts = dst_v[pl.ds(base + sub * L, L)]
                dstb_v[pl.ds(sub * L, L)] = dsts
                a_s = plsc.load_gather(asrc_v, [srcs])
                a_d = plsc.load_gather(adst_v, [dsts])
                s = a_s + a_d
                alpha = jnp.where(s > 0, s, 0.2 * s)
                p = jnp.exp(alpha - cvec)
                pbuf_v[pl.ds(base + sub * L, L)] = p
                for j in range(L):
                    sden_v[sub * L + j, pl.ds(0, L)] = \
                        p[jnp.full((L,), j, jnp.int32)]
            pltpu.sync_copy(sden_v, acc_sh.at[dstb_v], add=True)

        pltpu.sync_copy(pbuf_v, p_hbm.at[pl.ds(ebase + bi * IB, IB)])

    plsc.subcore_barrier()
    pltpu.sync_copy(acc_sh.at[pl.ds(sid * RPT, RPT)],
                    den_hbm.at[cid, pl.ds(sid * RPT, RPT)])


RCH = 80


@functools.partial(
    pl.kernel,
    compiler_params=_cp,
    out_type=jax.ShapeDtypeStruct((NC, NPAD, HID), jnp.float32),
    mesh=_mesh,
    scratch_types=[
        pltpu.VMEM((IB,), jnp.int32),
        pltpu.VMEM((IB,), jnp.int32),
        pltpu.VMEM((IB,), jnp.float32),
        pltpu.VMEM((RCH, HID), jnp.float32),
        pltpu.VMEM((RCH, HID), jnp.float32),
        pltpu.VMEM((RCH, HID), jnp.float32),
        pltpu.VMEM((RCH, HID), jnp.float32),
        pltpu.VMEM((RCH,), jnp.int32),
        pltpu.VMEM((RCH,), jnp.int32),
        pltpu.VMEM((8, HID), jnp.float32),
        pltpu.VMEM_SHARED((NPAD, HID), jnp.float32),
        pltpu.SemaphoreType.DMA,
        pltpu.SemaphoreType.DMA,
        pltpu.SemaphoreType.DMA,
        pltpu.SemaphoreType.DMA,
    ],
)
def _row_pass(p_hbm, src_hbm, dst_hbm, h_hbm, out_hbm,
              src_v, dst_v, pbuf_v, rows0_v, rows1_v, stage0_v, stage1_v,
              dstb0_v, dstb1_v, zero_v, acc_sh, gsem0, gsem1, ssem0, ssem1):
    cid = lax.axis_index("c")
    sid = lax.axis_index("s")
    wid = sid * NC + cid
    ebase = wid * EPW
    ICH = IB // RCH

    zvec = jnp.zeros((L,), jnp.float32)

    @pl.loop(0, 8)
    def _(r):
        for q in range(HID // L):
            zero_v[r, pl.ds(q * L, L)] = zvec

    @pl.loop(0, RPT // 8)
    def _(k):
        pltpu.sync_copy(zero_v, acc_sh.at[pl.ds(sid * RPT + k * 8, 8)])

    plsc.subcore_barrier()

    def step(ci, rows_v, stage_v, dstb_v, gsem, ssem):
        base = ci * RCH
        idx = src_v.at[pl.ds(base, RCH)]
        pltpu.make_async_copy(h_hbm.at[idx], rows_v, gsem).wait()

        @pl.when(ci >= 2)
        def _():
            pltpu.make_async_copy(stage_v, acc_sh.at[dstb_v], ssem).wait()

        for sub in range(RCH // L):
            dsts = dst_v[pl.ds(base + sub * L, L)]
            dstb_v[pl.ds(sub * L, L)] = dsts
            p = pbuf_v[pl.ds(base + sub * L, L)]
            for j in range(L):
                r = sub * L + j
                pj = p[jnp.full((L,), j, jnp.int32)]
                for q in range(HID // L):
                    stage_v[r, pl.ds(q * L, L)] = \
                        rows_v[r, pl.ds(q * L, L)] * pj
        pltpu.make_async_copy(stage_v, acc_sh.at[dstb_v], ssem).start(add=True)

        @pl.when(ci + 2 < ICH)
        def _():
            idx2 = src_v.at[pl.ds((ci + 2) * RCH, RCH)]
            pltpu.make_async_copy(h_hbm.at[idx2], rows_v, gsem).start()

    @pl.loop(0, NIB)
    def _(bi):
        pltpu.sync_copy(src_hbm.at[pl.ds(ebase + bi * IB, IB)], src_v)
        pltpu.sync_copy(dst_hbm.at[pl.ds(ebase + bi * IB, IB)], dst_v)
        pltpu.sync_copy(p_hbm.at[pl.ds(ebase + bi * IB, IB)], pbuf_v)

        pltpu.make_async_copy(
            h_hbm.at[src_v.at[pl.ds(0, RCH)]], rows0_v, gsem0).start()
        pltpu.make_async_copy(
            h_hbm.at[src_v.at[pl.ds(RCH, RCH)]], rows1_v, gsem1).start()

        @pl.loop(0, IB // RCH)
        def _(ci):
            @pl.when(ci % 2 == 0)
            def _():
                step(ci, rows0_v, stage0_v, dstb0_v, gsem0, ssem0)

            @pl.when(ci % 2 == 1)
            def _():
                step(ci, rows1_v, stage1_v, dstb1_v, gsem1, ssem1)

        pltpu.make_async_copy(stage0_v, acc_sh.at[dstb0_v], ssem0).wait()
        pltpu.make_async_copy(stage1_v, acc_sh.at[dstb1_v], ssem1).wait()

    plsc.subcore_barrier()
    pltpu.sync_copy(acc_sh.at[pl.ds(sid * RPT, RPT)],
                    out_hbm.at[cid, pl.ds(sid * RPT, RPT)])


def _lrelu(s):
    return jnp.where(s > 0, s, 0.2 * s)


def kernel(x, edge_index, W1, att_src1, att_dst1, b1,
           W2, att_src2, att_dst2, b2):
    src = edge_index[0]
    dst = edge_index[1]

    h1 = _matmul(x, W1)
    a_s1 = h1 @ att_src1
    a_d1 = h1 @ att_dst1
    c1 = jnp.max(a_s1) + jnp.max(a_d1)
    p_self1 = jnp.exp(_lrelu(a_s1 + a_d1) - c1)
    p1, den1 = _alpha_pass(a_s1, a_d1, jnp.full((L,), c1, jnp.float32),
                           src, dst)
    acc1 = _row_pass(p1, src, dst, h1)[:, :N]
    denom1 = den1[0, :N, 0] + den1[1, :N, 0] + p_self1
    num1 = acc1[0] + acc1[1] + p_self1[:, None] * h1
    r1 = jax.nn.relu(num1 / denom1[:, None] + b1)

    W2p = jnp.pad(W2, ((0, 0), (0, NCLSP - NCLS)))
    h2 = _matmul(r1, W2p)
    a_s2 = h2[:, :NCLS] @ att_src2
    a_d2 = h2[:, :NCLS] @ att_dst2
    c2 = jnp.max(a_s2) + jnp.max(a_d2)
    p_self2 = jnp.exp(_lrelu(a_s2 + a_d2) - c2)
    acc2 = _edge_pass_cls(a_s2, a_d2, jnp.full((L,), c2, jnp.float32),
                          src, dst, h2)[:, :N]
    denom2 = acc2[0, :, 0] + acc2[1, :, 0] + p_self2
    num2 = (acc2[0, :, L:L + NCLS] + acc2[1, :, L:L + NCLS]
            + p_self2[:, None] * h2[:, :NCLS])
    return num2 / denom2[:, None] + b2


# speedup vs baseline: 55.1660x; 1.0071x over previous
import dataclasses
import functools

import jax
import jax.numpy as jnp
from jax import lax
from jax.experimental import pallas as pl
from jax.experimental.pallas import tpu as pltpu
from jax.experimental.pallas import tpu_sc as plsc

N = 10000
E = 320000
F_IN = 128
HID = 128
NCLS = 40
NCLSP = 48

NC = 2
NS = 16
L = 16
NW = NC * NS
EPW = E // NW
NPAD = 10240
RPT = NPAD // NS
IB = 2000
NIB = EPW // IB
ZR = 16


def _matmul(x, w):
    m, k = x.shape
    _, n = w.shape
    bm = 2000

    def body(x_ref, w_ref, o_ref):
        o_ref[...] = jnp.dot(x_ref[...], w_ref[...],
                             preferred_element_type=jnp.float32)

    return pl.pallas_call(
        body,
        grid=(m // bm,),
        in_specs=[pl.BlockSpec((bm, k), lambda i: (i, 0)),
                  pl.BlockSpec((k, n), lambda i: (0, 0))],
        out_specs=pl.BlockSpec((bm, n), lambda i: (i, 0)),
        out_shape=jax.ShapeDtypeStruct((m, n), jnp.float32),
    )(x, w)


def _make_edge_pass(D, CH):
    W = D + L
    ICH = IB // CH
    mesh = plsc.VectorSubcoreMesh(core_axis_name="c", subcore_axis_name="s")
    cp = pltpu.CompilerParams()
    if "needs_layout_passes" in pltpu.CompilerParams.__dataclass_fields__:
        cp = dataclasses.replace(cp, needs_layout_passes=False)
    if "use_tc_tiling_on_sc" in pltpu.CompilerParams.__dataclass_fields__:
        cp = dataclasses.replace(cp, use_tc_tiling_on_sc=False)

    @functools.partial(
        pl.kernel,
        compiler_params=cp,
        out_type=jax.ShapeDtypeStruct((NC, NPAD, W), jnp.float32),
        mesh=mesh,
        scratch_types=[
            pltpu.VMEM((IB,), jnp.int32),
            pltpu.VMEM((IB,), jnp.int32),
            pltpu.VMEM((N,), jnp.float32),
            pltpu.VMEM((N,), jnp.float32),
            pltpu.VMEM((L,), jnp.float32),
            pltpu.VMEM((CH, D), jnp.float32),
            pltpu.VMEM((CH, D), jnp.float32),
            pltpu.VMEM((CH, W), jnp.float32),
            pltpu.VMEM((CH, W), jnp.float32),
            pltpu.VMEM((CH,), jnp.int32),
            pltpu.VMEM((CH,), jnp.int32),
            pltpu.VMEM((ZR, W), jnp.float32),
            pltpu.VMEM_SHARED((NPAD, W), jnp.float32),
            pltpu.SemaphoreType.DMA,
            pltpu.SemaphoreType.DMA,
            pltpu.SemaphoreType.DMA,
            pltpu.SemaphoreType.DMA,
        ],
    )
    def edge_pass(asrc_hbm, adst_hbm, c_hbm, src_hbm, dst_hbm, h_hbm, out_hbm,
                  src_v, dst_v, asrc_v, adst_v, c_v, rows0_v, rows1_v,
                  stage0_v, stage1_v, dstb0_v, dstb1_v, zero_v, acc_sh,
                  gsem0, gsem1, ssem0, ssem1):
        cid = lax.axis_index("c")
        sid = lax.axis_index("s")
        wid = sid * NC + cid
        ebase = wid * EPW

        pltpu.sync_copy(asrc_hbm, asrc_v)
        pltpu.sync_copy(adst_hbm, adst_v)
        pltpu.sync_copy(c_hbm, c_v)

        zvec = jnp.zeros((L,), jnp.float32)

        @pl.loop(0, ZR)
        def _(r):
            for q in range(W // L):
                zero_v[r, pl.ds(q * L, L)] = zvec

        @pl.loop(0, RPT // ZR)
        def _(k):
            pltpu.sync_copy(zero_v, acc_sh.at[pl.ds(sid * RPT + k * ZR, ZR)])

        plsc.subcore_barrier()

        cvec = c_v[...]

        def step(ci, rows_v, stage_v, dstb_v, gsem, ssem):
            base = ci * CH
            idx = src_v.at[pl.ds(base, CH)]
            pltpu.make_async_copy(h_hbm.at[idx], rows_v, gsem).wait()

            @pl.when(ci >= 2)
            def _():
                pltpu.make_async_copy(
                    stage_v, acc_sh.at[dstb_v], ssem).wait()

            for sub in range(CH // L):
                srcs = src_v[pl.ds(base + sub * L, L)]
                dsts = dst_v[pl.ds(base + sub * L, L)]
                dstb_v[pl.ds(sub * L, L)] = dsts
                a_s = plsc.load_gather(asrc_v, [srcs])
                a_d = plsc.load_gather(adst_v, [dsts])
                s = a_s + a_d
                alpha = jnp.where(s > 0, s, 0.2 * s)
                p = jnp.exp(alpha - cvec)
                for j in range(L):
                    r = sub * L + j
                    pj = p[jnp.full((L,), j, jnp.int32)]
                    stage_v[r, pl.ds(0, L)] = pj
                    for q in range(D // L):
                        stage_v[r, pl.ds(L + q * L, L)] = \
                            rows_v[r, pl.ds(q * L, L)] * pj
            scat = pltpu.make_async_copy(stage_v, acc_sh.at[dstb_v], ssem)
            scat.start(add=True)

            @pl.when(ci + 2 < ICH)
            def _():
                idx2 = src_v.at[pl.ds((ci + 2) * CH, CH)]
                pltpu.make_async_copy(h_hbm.at[idx2], rows_v, gsem).start()

        @pl.loop(0, NIB)
        def _(bi):
            pltpu.sync_copy(src_hbm.at[pl.ds(ebase + bi * IB, IB)], src_v)
            pltpu.sync_copy(dst_hbm.at[pl.ds(ebase + bi * IB, IB)], dst_v)

            pltpu.make_async_copy(
                h_hbm.at[src_v.at[pl.ds(0, CH)]], rows0_v, gsem0).start()
            pltpu.make_async_copy(
                h_hbm.at[src_v.at[pl.ds(CH, CH)]], rows1_v, gsem1).start()

            @pl.loop(0, ICH)
            def _(ci):
                @pl.when(ci % 2 == 0)
                def _():
                    step(ci, rows0_v, stage0_v, dstb0_v, gsem0, ssem0)

                @pl.when(ci % 2 == 1)
                def _():
                    step(ci, rows1_v, stage1_v, dstb1_v, gsem1, ssem1)

            pltpu.make_async_copy(stage0_v, acc_sh.at[dstb0_v], ssem0).wait()
            pltpu.make_async_copy(stage1_v, acc_sh.at[dstb1_v], ssem1).wait()

        plsc.subcore_barrier()
        pltpu.sync_copy(acc_sh.at[pl.ds(sid * RPT, RPT)],
                        out_hbm.at[cid, pl.ds(sid * RPT, RPT)])

    return edge_pass


_edge_pass_cls = _make_edge_pass(NCLSP, 80)

_mesh = plsc.VectorSubcoreMesh(core_axis_name="c", subcore_axis_name="s")
_cp = pltpu.CompilerParams()
if "needs_layout_passes" in pltpu.CompilerParams.__dataclass_fields__:
    _cp = dataclasses.replace(_cp, needs_layout_passes=False)
if "use_tc_tiling_on_sc" in pltpu.CompilerParams.__dataclass_fields__:
    _cp = dataclasses.replace(_cp, use_tc_tiling_on_sc=False)

ACH = 80


@functools.partial(
    pl.kernel,
    compiler_params=_cp,
    out_type=[jax.ShapeDtypeStruct((E,), jnp.float32),
              jax.ShapeDtypeStruct((NC, NPAD, L), jnp.float32)],
    mesh=_mesh,
    scratch_types=[
        pltpu.VMEM((IB,), jnp.int32),
        pltpu.VMEM((IB,), jnp.int32),
        pltpu.VMEM((N,), jnp.float32),
        pltpu.VMEM((N,), jnp.float32),
        pltpu.VMEM((L,), jnp.float32),
        pltpu.VMEM((IB,), jnp.float32),
        pltpu.VMEM((ACH, L), jnp.float32),
        pltpu.VMEM((ACH, L), jnp.float32),
        pltpu.VMEM((ACH,), jnp.int32),
        pltpu.VMEM((ACH,), jnp.int32),
        pltpu.VMEM((ZR, L), jnp.float32),
        pltpu.VMEM_SHARED((NPAD, L), jnp.float32),
        pltpu.SemaphoreType.DMA,
        pltpu.SemaphoreType.DMA,
    ],
)
def _alpha_pass(asrc_hbm, adst_hbm, c_hbm, src_hbm, dst_hbm, p_hbm, den_hbm,
                src_v, dst_v, asrc_v, adst_v, c_v, pbuf_v, sden0_v, sden1_v,
                dstb0_v, dstb1_v, zero_v, acc_sh, ssem0, ssem1):
    cid = lax.axis_index("c")
    sid = lax.axis_index("s")
    wid = sid * NC + cid
    ebase = wid * EPW

    pltpu.sync_copy(asrc_hbm, asrc_v)
    pltpu.sync_copy(adst_hbm, adst_v)
    pltpu.sync_copy(c_hbm, c_v)

    zvec = jnp.zeros((L,), jnp.float32)

    @pl.loop(0, ZR)
    def _(r):
        zero_v[r, pl.ds(0, L)] = zvec

    @pl.loop(0, RPT // ZR)
    def _(k):
        pltpu.sync_copy(zero_v, acc_sh.at[pl.ds(sid * RPT + k * ZR, ZR)])

    plsc.subcore_barrier()

    cvec = c_v[...]

    def astep(ci, sden_v, dstb_v, ssem):
        base = ci * ACH

        @pl.when(ci >= 2)
        def _():
            pltpu.make_async_copy(sden_v, acc_sh.at[dstb_v], ssem).wait()

        for sub in range(ACH // L):
            srcs = src_v[pl.ds(base + sub * L, L)]
            dsts = dst_v[pl.ds(base + sub * L, L)]
            dstb_v[pl.ds(sub * L, L)] = dsts
            a_s = plsc.load_gather(asrc_v, [srcs])
            a_d = plsc.load_gather(adst_v, [dsts])
            s = a_s + a_d
            alpha = jnp.where(s > 0, s, 0.2 * s)
            p = jnp.exp(alpha - cvec)
            pbuf_v[pl.ds(base + sub * L, L)] = p
            for j in range(L):
                sden_v[sub * L + j, pl.ds(0, L)] = \
                    p[jnp.full((L,), j, jnp.int32)]
        pltpu.make_async_copy(sden_v, acc_sh.at[dstb_v], ssem).start(add=True)

    @pl.loop(0, NIB)
    def _(bi):
        pltpu.sync_copy(src_hbm.at[pl.ds(ebase + bi * IB, IB)], src_v)
        pltpu.sync_copy(dst_hbm.at[pl.ds(ebase + bi * IB, IB)], dst_v)

        @pl.loop(0, IB // ACH)
        def _(ci):
            @pl.when(ci % 2 == 0)
            def _():
                astep(ci, sden0_v, dstb0_v, ssem0)

            @pl.when(ci % 2 == 1)
            def _():
                astep(ci, sden1_v, dstb1_v, ssem1)

        pltpu.make_async_copy(sden0_v, acc_sh.at[dstb0_v], ssem0).wait()
        pltpu.make_async_copy(sden1_v, acc_sh.at[dstb1_v], ssem1).wait()
        pltpu.sync_copy(pbuf_v, p_hbm.at[pl.ds(ebase + bi * IB, IB)])

    plsc.subcore_barrier()
    pltpu.sync_copy(acc_sh.at[pl.ds(sid * RPT, RPT)],
                    den_hbm.at[cid, pl.ds(sid * RPT, RPT)])


RCH = 80


@functools.partial(
    pl.kernel,
    compiler_params=_cp,
    out_type=jax.ShapeDtypeStruct((NC, NPAD, HID), jnp.float32),
    mesh=_mesh,
    scratch_types=[
        pltpu.VMEM((IB,), jnp.int32),
        pltpu.VMEM((IB,), jnp.int32),
        pltpu.VMEM((IB,), jnp.float32),
        pltpu.VMEM((RCH, HID), jnp.float32),
        pltpu.VMEM((RCH, HID), jnp.float32),
        pltpu.VMEM((RCH, HID), jnp.float32),
        pltpu.VMEM((RCH, HID), jnp.float32),
        pltpu.VMEM((RCH,), jnp.int32),
        pltpu.VMEM((RCH,), jnp.int32),
        pltpu.VMEM((8, HID), jnp.float32),
        pltpu.VMEM_SHARED((NPAD, HID), jnp.float32),
        pltpu.SemaphoreType.DMA,
        pltpu.SemaphoreType.DMA,
        pltpu.SemaphoreType.DMA,
        pltpu.SemaphoreType.DMA,
    ],
)
def _row_pass(p_hbm, src_hbm, dst_hbm, h_hbm, out_hbm,
              src_v, dst_v, pbuf_v, rows0_v, rows1_v, stage0_v, stage1_v,
              dstb0_v, dstb1_v, zero_v, acc_sh, gsem0, gsem1, ssem0, ssem1):
    cid = lax.axis_index("c")
    sid = lax.axis_index("s")
    wid = sid * NC + cid
    ebase = wid * EPW
    ICH = IB // RCH

    zvec = jnp.zeros((L,), jnp.float32)

    @pl.loop(0, 8)
    def _(r):
        for q in range(HID // L):
            zero_v[r, pl.ds(q * L, L)] = zvec

    @pl.loop(0, RPT // 8)
    def _(k):
        pltpu.sync_copy(zero_v, acc_sh.at[pl.ds(sid * RPT + k * 8, 8)])

    plsc.subcore_barrier()

    def step(ci, rows_v, stage_v, dstb_v, gsem, ssem):
        base = ci * RCH
        idx = src_v.at[pl.ds(base, RCH)]
        pltpu.make_async_copy(h_hbm.at[idx], rows_v, gsem).wait()

        @pl.when(ci >= 2)
        def _():
            pltpu.make_async_copy(stage_v, acc_sh.at[dstb_v], ssem).wait()

        for sub in range(RCH // L):
            dsts = dst_v[pl.ds(base + sub * L, L)]
            dstb_v[pl.ds(sub * L, L)] = dsts
            p = pbuf_v[pl.ds(base + sub * L, L)]
            for j in range(L):
                r = sub * L + j
                pj = p[jnp.full((L,), j, jnp.int32)]
                for q in range(HID // L):
                    stage_v[r, pl.ds(q * L, L)] = \
                        rows_v[r, pl.ds(q * L, L)] * pj
        pltpu.make_async_copy(stage_v, acc_sh.at[dstb_v], ssem).start(add=True)

        @pl.when(ci + 2 < ICH)
        def _():
            idx2 = src_v.at[pl.ds((ci + 2) * RCH, RCH)]
            pltpu.make_async_copy(h_hbm.at[idx2], rows_v, gsem).start()

    @pl.loop(0, NIB)
    def _(bi):
        pltpu.sync_copy(src_hbm.at[pl.ds(ebase + bi * IB, IB)], src_v)
        pltpu.sync_copy(dst_hbm.at[pl.ds(ebase + bi * IB, IB)], dst_v)
        pltpu.sync_copy(p_hbm.at[pl.ds(ebase + bi * IB, IB)], pbuf_v)

        pltpu.make_async_copy(
            h_hbm.at[src_v.at[pl.ds(0, RCH)]], rows0_v, gsem0).start()
        pltpu.make_async_copy(
            h_hbm.at[src_v.at[pl.ds(RCH, RCH)]], rows1_v, gsem1).start()

        @pl.loop(0, IB // RCH)
        def _(ci):
            @pl.when(ci % 2 == 0)
            def _():
                step(ci, rows0_v, stage0_v, dstb0_v, gsem0, ssem0)

            @pl.when(ci % 2 == 1)
            def _():
                step(ci, rows1_v, stage1_v, dstb1_v, gsem1, ssem1)

        pltpu.make_async_copy(stage0_v, acc_sh.at[dstb0_v], ssem0).wait()
        pltpu.make_async_copy(stage1_v, acc_sh.at[dstb1_v], ssem1).wait()

    plsc.subcore_barrier()
    pltpu.sync_copy(acc_sh.at[pl.ds(sid * RPT, RPT)],
                    out_hbm.at[cid, pl.ds(sid * RPT, RPT)])


def _lrelu(s):
    return jnp.where(s > 0, s, 0.2 * s)


def kernel(x, edge_index, W1, att_src1, att_dst1, b1,
           W2, att_src2, att_dst2, b2):
    src = edge_index[0]
    dst = edge_index[1]

    h1 = _matmul(x, W1)
    a_s1 = h1 @ att_src1
    a_d1 = h1 @ att_dst1
    c1 = jnp.max(a_s1) + jnp.max(a_d1)
    p_self1 = jnp.exp(_lrelu(a_s1 + a_d1) - c1)
    p1, den1 = _alpha_pass(a_s1, a_d1, jnp.full((L,), c1, jnp.float32),
                           src, dst)
    acc1 = _row_pass(p1, src, dst, h1)[:, :N]
    denom1 = den1[0, :N, 0] + den1[1, :N, 0] + p_self1
    num1 = acc1[0] + acc1[1] + p_self1[:, None] * h1
    r1 = jax.nn.relu(num1 / denom1[:, None] + b1)

    W2p = jnp.pad(W2, ((0, 0), (0, NCLSP - NCLS)))
    h2 = _matmul(r1, W2p)
    a_s2 = h2[:, :NCLS] @ att_src2
    a_d2 = h2[:, :NCLS] @ att_dst2
    c2 = jnp.max(a_s2) + jnp.max(a_d2)
    p_self2 = jnp.exp(_lrelu(a_s2 + a_d2) - c2)
    acc2 = _edge_pass_cls(a_s2, a_d2, jnp.full((L,), c2, jnp.float32),
                          src, dst, h2)[:, :N]
    denom2 = acc2[0, :, 0] + acc2[1, :, 0] + p_self2
    num2 = (acc2[0, :, L:L + NCLS] + acc2[1, :, L:L + NCLS]
            + p_self2[:, None] * h2[:, :NCLS])
    return num2 / denom2[:, None] + b2


# speedup vs baseline: 55.2675x; 1.0018x over previous
import dataclasses
import functools

import jax
import jax.numpy as jnp
from jax import lax
from jax.experimental import pallas as pl
from jax.experimental.pallas import tpu as pltpu
from jax.experimental.pallas import tpu_sc as plsc

N = 10000
E = 320000
F_IN = 128
HID = 128
NCLS = 40
NCLSP = 48

NC = 2
NS = 16
L = 16
NW = NC * NS
EPW = E // NW
NPAD = 10240
RPT = NPAD // NS
IB = 2000
NIB = EPW // IB
ZR = 16


def _matmul(x, w):
    m, k = x.shape
    _, n = w.shape
    bm = 2000

    def body(x_ref, w_ref, o_ref):
        o_ref[...] = jnp.dot(x_ref[...], w_ref[...],
                             preferred_element_type=jnp.float32)

    return pl.pallas_call(
        body,
        grid=(m // bm,),
        in_specs=[pl.BlockSpec((bm, k), lambda i: (i, 0)),
                  pl.BlockSpec((k, n), lambda i: (0, 0))],
        out_specs=pl.BlockSpec((bm, n), lambda i: (i, 0)),
        out_shape=jax.ShapeDtypeStruct((m, n), jnp.float32),
    )(x, w)


def _make_edge_pass(D, CH):
    W = D
    ICH = IB // CH
    mesh = plsc.VectorSubcoreMesh(core_axis_name="c", subcore_axis_name="s")
    cp = pltpu.CompilerParams()
    if "needs_layout_passes" in pltpu.CompilerParams.__dataclass_fields__:
        cp = dataclasses.replace(cp, needs_layout_passes=False)
    if "use_tc_tiling_on_sc" in pltpu.CompilerParams.__dataclass_fields__:
        cp = dataclasses.replace(cp, use_tc_tiling_on_sc=False)

    @functools.partial(
        pl.kernel,
        compiler_params=cp,
        out_type=[jax.ShapeDtypeStruct((NC, NPAD, W), jnp.float32),
                  jax.ShapeDtypeStruct((NC, NS, N), jnp.float32)],
        mesh=mesh,
        scratch_types=[
            pltpu.VMEM((IB,), jnp.int32),
            pltpu.VMEM((IB,), jnp.int32),
            pltpu.VMEM((N,), jnp.float32),
            pltpu.VMEM((N,), jnp.float32),
            pltpu.VMEM((L,), jnp.float32),
            pltpu.VMEM((CH, D), jnp.float32),
            pltpu.VMEM((CH, D), jnp.float32),
            pltpu.VMEM((CH, W), jnp.float32),
            pltpu.VMEM((CH, W), jnp.float32),
            pltpu.VMEM((CH,), jnp.int32),
            pltpu.VMEM((CH,), jnp.int32),
            pltpu.VMEM((N,), jnp.float32),
            pltpu.VMEM((ZR, W), jnp.float32),
            pltpu.VMEM_SHARED((NPAD, W), jnp.float32),
            pltpu.SemaphoreType.DMA,
            pltpu.SemaphoreType.DMA,
            pltpu.SemaphoreType.DMA,
            pltpu.SemaphoreType.DMA,
        ],
    )
    def edge_pass(asrc_hbm, adst_hbm, c_hbm, src_hbm, dst_hbm, h_hbm, out_hbm,
                  den_hbm, src_v, dst_v, asrc_v, adst_v, c_v, rows0_v, rows1_v,
                  stage0_v, stage1_v, dstb0_v, dstb1_v, den_v, zero_v, acc_sh,
                  gsem0, gsem1, ssem0, ssem1):
        cid = lax.axis_index("c")
        sid = lax.axis_index("s")
        wid = sid * NC + cid
        ebase = wid * EPW

        pltpu.sync_copy(asrc_hbm, asrc_v)
        pltpu.sync_copy(adst_hbm, adst_v)
        pltpu.sync_copy(c_hbm, c_v)

        zvec = jnp.zeros((L,), jnp.float32)

        @pl.loop(0, ZR)
        def _(r):
            for q in range(W // L):
                zero_v[r, pl.ds(q * L, L)] = zvec

        @pl.loop(0, RPT // ZR)
        def _(k):
            pltpu.sync_copy(zero_v, acc_sh.at[pl.ds(sid * RPT + k * ZR, ZR)])

        @pl.loop(0, N // L)
        def _(r):
            den_v[pl.ds(r * L, L)] = zvec

        plsc.subcore_barrier()

        cvec = c_v[...]

        def step(ci, rows_v, stage_v, dstb_v, gsem, ssem):
            base = ci * CH
            idx = src_v.at[pl.ds(base, CH)]
            pltpu.make_async_copy(h_hbm.at[idx], rows_v, gsem).wait()

            @pl.when(ci >= 2)
            def _():
                pltpu.make_async_copy(
                    stage_v, acc_sh.at[dstb_v], ssem).wait()

            for sub in range(CH // L):
                srcs = src_v[pl.ds(base + sub * L, L)]
                dsts = dst_v[pl.ds(base + sub * L, L)]
                dstb_v[pl.ds(sub * L, L)] = dsts
                a_s = plsc.load_gather(asrc_v, [srcs])
                a_d = plsc.load_gather(adst_v, [dsts])
                s = a_s + a_d
                alpha = jnp.where(s > 0, s, 0.2 * s)
                p = jnp.exp(alpha - cvec)
                plsc.addupdate_scatter(den_v, [dsts], p)
                for j in range(L):
                    r = sub * L + j
                    pj = p[jnp.full((L,), j, jnp.int32)]
                    for q in range(D // L):
                        stage_v[r, pl.ds(q * L, L)] = \
                            rows_v[r, pl.ds(q * L, L)] * pj
            scat = pltpu.make_async_copy(stage_v, acc_sh.at[dstb_v], ssem)
            scat.start(add=True)

            @pl.when(ci + 2 < ICH)
            def _():
                idx2 = src_v.at[pl.ds((ci + 2) * CH, CH)]
                pltpu.make_async_copy(h_hbm.at[idx2], rows_v, gsem).start()

        @pl.loop(0, NIB)
        def _(bi):
            pltpu.sync_copy(src_hbm.at[pl.ds(ebase + bi * IB, IB)], src_v)
            pltpu.sync_copy(dst_hbm.at[pl.ds(ebase + bi * IB, IB)], dst_v)

            pltpu.make_async_copy(
                h_hbm.at[src_v.at[pl.ds(0, CH)]], rows0_v, gsem0).start()
            pltpu.make_async_copy(
                h_hbm.at[src_v.at[pl.ds(CH, CH)]], rows1_v, gsem1).start()

            @pl.loop(0, ICH)
            def _(ci):
                @pl.when(ci % 2 == 0)
                def _():
                    step(ci, rows0_v, stage0_v, dstb0_v, gsem0, ssem0)

                @pl.when(ci % 2 == 1)
                def _():
                    step(ci, rows1_v, stage1_v, dstb1_v, gsem1, ssem1)

            pltpu.make_async_copy(stage0_v, acc_sh.at[dstb0_v], ssem0).wait()
            pltpu.make_async_copy(stage1_v, acc_sh.at[dstb1_v], ssem1).wait()

        pltpu.sync_copy(den_v, den_hbm.at[cid, sid])
        plsc.subcore_barrier()
        pltpu.sync_copy(acc_sh.at[pl.ds(sid * RPT, RPT)],
                        out_hbm.at[cid, pl.ds(sid * RPT, RPT)])

    return edge_pass


_edge_pass_cls = _make_edge_pass(NCLSP, 80)

_mesh = plsc.VectorSubcoreMesh(core_axis_name="c", subcore_axis_name="s")
_cp = pltpu.CompilerParams()
if "needs_layout_passes" in pltpu.CompilerParams.__dataclass_fields__:
    _cp = dataclasses.replace(_cp, needs_layout_passes=False)
if "use_tc_tiling_on_sc" in pltpu.CompilerParams.__dataclass_fields__:
    _cp = dataclasses.replace(_cp, use_tc_tiling_on_sc=False)

ACH = 80


@functools.partial(
    pl.kernel,
    compiler_params=_cp,
    out_type=[jax.ShapeDtypeStruct((E,), jnp.float32),
              jax.ShapeDtypeStruct((NC, NPAD, L), jnp.float32)],
    mesh=_mesh,
    scratch_types=[
        pltpu.VMEM((IB,), jnp.int32),
        pltpu.VMEM((IB,), jnp.int32),
        pltpu.VMEM((N,), jnp.float32),
        pltpu.VMEM((N,), jnp.float32),
        pltpu.VMEM((L,), jnp.float32),
        pltpu.VMEM((IB,), jnp.float32),
        pltpu.VMEM((ACH, L), jnp.float32),
        pltpu.VMEM((ACH, L), jnp.float32),
        pltpu.VMEM((ACH,), jnp.int32),
        pltpu.VMEM((ACH,), jnp.int32),
        pltpu.VMEM((ZR, L), jnp.float32),
        pltpu.VMEM_SHARED((NPAD, L), jnp.float32),
        pltpu.SemaphoreType.DMA,
        pltpu.SemaphoreType.DMA,
    ],
)
def _alpha_pass(asrc_hbm, adst_hbm, c_hbm, src_hbm, dst_hbm, p_hbm, den_hbm,
                src_v, dst_v, asrc_v, adst_v, c_v, pbuf_v, sden0_v, sden1_v,
                dstb0_v, dstb1_v, zero_v, acc_sh, ssem0, ssem1):
    cid = lax.axis_index("c")
    sid = lax.axis_index("s")
    wid = sid * NC + cid
    ebase = wid * EPW

    pltpu.sync_copy(asrc_hbm, asrc_v)
    pltpu.sync_copy(adst_hbm, adst_v)
    pltpu.sync_copy(c_hbm, c_v)

    zvec = jnp.zeros((L,), jnp.float32)

    @pl.loop(0, ZR)
    def _(r):
        zero_v[r, pl.ds(0, L)] = zvec

    @pl.loop(0, RPT // ZR)
    def _(k):
        pltpu.sync_copy(zero_v, acc_sh.at[pl.ds(sid * RPT + k * ZR, ZR)])

    plsc.subcore_barrier()

    cvec = c_v[...]

    def astep(ci, sden_v, dstb_v, ssem):
        base = ci * ACH

        @pl.when(ci >= 2)
        def _():
            pltpu.make_async_copy(sden_v, acc_sh.at[dstb_v], ssem).wait()

        for sub in range(ACH // L):
            srcs = src_v[pl.ds(base + sub * L, L)]
            dsts = dst_v[pl.ds(base + sub * L, L)]
            dstb_v[pl.ds(sub * L, L)] = dsts
            a_s = plsc.load_gather(asrc_v, [srcs])
            a_d = plsc.load_gather(adst_v, [dsts])
            s = a_s + a_d
            alpha = jnp.where(s > 0, s, 0.2 * s)
            p = jnp.exp(alpha - cvec)
            pbuf_v[pl.ds(base + sub * L, L)] = p
            for j in range(L):
                sden_v[sub * L + j, pl.ds(0, L)] = \
                    p[jnp.full((L,), j, jnp.int32)]
        pltpu.make_async_copy(sden_v, acc_sh.at[dstb_v], ssem).start(add=True)

    @pl.loop(0, NIB)
    def _(bi):
        pltpu.sync_copy(src_hbm.at[pl.ds(ebase + bi * IB, IB)], src_v)
        pltpu.sync_copy(dst_hbm.at[pl.ds(ebase + bi * IB, IB)], dst_v)

        @pl.loop(0, IB // ACH)
        def _(ci):
            @pl.when(ci % 2 == 0)
            def _():
                astep(ci, sden0_v, dstb0_v, ssem0)

            @pl.when(ci % 2 == 1)
            def _():
                astep(ci, sden1_v, dstb1_v, ssem1)

        pltpu.make_async_copy(sden0_v, acc_sh.at[dstb0_v], ssem0).wait()
        pltpu.make_async_copy(sden1_v, acc_sh.at[dstb1_v], ssem1).wait()
        pltpu.sync_copy(pbuf_v, p_hbm.at[pl.ds(ebase + bi * IB, IB)])

    plsc.subcore_barrier()
    pltpu.sync_copy(acc_sh.at[pl.ds(sid * RPT, RPT)],
                    den_hbm.at[cid, pl.ds(sid * RPT, RPT)])


RCH = 80


@functools.partial(
    pl.kernel,
    compiler_params=_cp,
    out_type=jax.ShapeDtypeStruct((NC, NPAD, HID), jnp.float32),
    mesh=_mesh,
    scratch_types=[
        pltpu.VMEM((IB,), jnp.int32),
        pltpu.VMEM((IB,), jnp.int32),
        pltpu.VMEM((IB,), jnp.float32),
        pltpu.VMEM((RCH, HID), jnp.float32),
        pltpu.VMEM((RCH, HID), jnp.float32),
        pltpu.VMEM((RCH, HID), jnp.float32),
        pltpu.VMEM((RCH, HID), jnp.float32),
        pltpu.VMEM((RCH,), jnp.int32),
        pltpu.VMEM((RCH,), jnp.int32),
        pltpu.VMEM((8, HID), jnp.float32),
        pltpu.VMEM_SHARED((NPAD, HID), jnp.float32),
        pltpu.SemaphoreType.DMA,
        pltpu.SemaphoreType.DMA,
        pltpu.SemaphoreType.DMA,
        pltpu.SemaphoreType.DMA,
    ],
)
def _row_pass(p_hbm, src_hbm, dst_hbm, h_hbm, out_hbm,
              src_v, dst_v, pbuf_v, rows0_v, rows1_v, stage0_v, stage1_v,
              dstb0_v, dstb1_v, zero_v, acc_sh, gsem0, gsem1, ssem0, ssem1):
    cid = lax.axis_index("c")
    sid = lax.axis_index("s")
    wid = sid * NC + cid
    ebase = wid * EPW
    ICH = IB // RCH

    zvec = jnp.zeros((L,), jnp.float32)

    @pl.loop(0, 8)
    def _(r):
        for q in range(HID // L):
            zero_v[r, pl.ds(q * L, L)] = zvec

    @pl.loop(0, RPT // 8)
    def _(k):
        pltpu.sync_copy(zero_v, acc_sh.at[pl.ds(sid * RPT + k * 8, 8)])

    plsc.subcore_barrier()

    def step(ci, rows_v, stage_v, dstb_v, gsem, ssem):
        base = ci * RCH
        idx = src_v.at[pl.ds(base, RCH)]
        pltpu.make_async_copy(h_hbm.at[idx], rows_v, gsem).wait()

        @pl.when(ci >= 2)
        def _():
            pltpu.make_async_copy(stage_v, acc_sh.at[dstb_v], ssem).wait()

        for sub in range(RCH // L):
            dsts = dst_v[pl.ds(base + sub * L, L)]
            dstb_v[pl.ds(sub * L, L)] = dsts
            p = pbuf_v[pl.ds(base + sub * L, L)]
            for j in range(L):
                r = sub * L + j
                pj = p[jnp.full((L,), j, jnp.int32)]
                for q in range(HID // L):
                    stage_v[r, pl.ds(q * L, L)] = \
                        rows_v[r, pl.ds(q * L, L)] * pj
        pltpu.make_async_copy(stage_v, acc_sh.at[dstb_v], ssem).start(add=True)

        @pl.when(ci + 2 < ICH)
        def _():
            idx2 = src_v.at[pl.ds((ci + 2) * RCH, RCH)]
            pltpu.make_async_copy(h_hbm.at[idx2], rows_v, gsem).start()

    @pl.loop(0, NIB)
    def _(bi):
        pltpu.sync_copy(src_hbm.at[pl.ds(ebase + bi * IB, IB)], src_v)
        pltpu.sync_copy(dst_hbm.at[pl.ds(ebase + bi * IB, IB)], dst_v)
        pltpu.sync_copy(p_hbm.at[pl.ds(ebase + bi * IB, IB)], pbuf_v)

        pltpu.make_async_copy(
            h_hbm.at[src_v.at[pl.ds(0, RCH)]], rows0_v, gsem0).start()
        pltpu.make_async_copy(
            h_hbm.at[src_v.at[pl.ds(RCH, RCH)]], rows1_v, gsem1).start()

        @pl.loop(0, IB // RCH)
        def _(ci):
            @pl.when(ci % 2 == 0)
            def _():
                step(ci, rows0_v, stage0_v, dstb0_v, gsem0, ssem0)

            @pl.when(ci % 2 == 1)
            def _():
                step(ci, rows1_v, stage1_v, dstb1_v, gsem1, ssem1)

        pltpu.make_async_copy(stage0_v, acc_sh.at[dstb0_v], ssem0).wait()
        pltpu.make_async_copy(stage1_v, acc_sh.at[dstb1_v], ssem1).wait()

    plsc.subcore_barrier()
    pltpu.sync_copy(acc_sh.at[pl.ds(sid * RPT, RPT)],
                    out_hbm.at[cid, pl.ds(sid * RPT, RPT)])


def _lrelu(s):
    return jnp.where(s > 0, s, 0.2 * s)


def kernel(x, edge_index, W1, att_src1, att_dst1, b1,
           W2, att_src2, att_dst2, b2):
    src = edge_index[0]
    dst = edge_index[1]

    h1 = _matmul(x, W1)
    a_s1 = h1 @ att_src1
    a_d1 = h1 @ att_dst1
    c1 = jnp.max(a_s1) + jnp.max(a_d1)
    p_self1 = jnp.exp(_lrelu(a_s1 + a_d1) - c1)
    p1, den1 = _alpha_pass(a_s1, a_d1, jnp.full((L,), c1, jnp.float32),
                           src, dst)
    acc1 = _row_pass(p1, src, dst, h1)[:, :N]
    denom1 = den1[0, :N, 0] + den1[1, :N, 0] + p_self1
    num1 = acc1[0] + acc1[1] + p_self1[:, None] * h1
    r1 = jax.nn.relu(num1 / denom1[:, None] + b1)

    W2p = jnp.pad(W2, ((0, 0), (0, NCLSP - NCLS)))
    h2 = _matmul(r1, W2p)
    a_s2 = h2[:, :NCLS] @ att_src2
    a_d2 = h2[:, :NCLS] @ att_dst2
    c2 = jnp.max(a_s2) + jnp.max(a_d2)
    p_self2 = jnp.exp(_lrelu(a_s2 + a_d2) - c2)
    acc2, den2 = _edge_pass_cls(a_s2, a_d2, jnp.full((L,), c2, jnp.float32),
                                src, dst, h2)
    acc2 = acc2[:, :N]
    denom2 = jnp.sum(den2, axis=(0, 1)) + p_self2
    num2 = (acc2[0, :, :NCLS] + acc2[1, :, :NCLS]
            + p_self2[:, None] * h2[:, :NCLS])
    return num2 / denom2[:, None] + b2
